# Initial kernel scaffold; baseline (speedup 1.0000x reference)
import jax
import jax.numpy as jnp
from jax.experimental import pallas as pl


def kernel(x_in, edge_index, Wl1, bl1, Wr1, Wl2, bl2, Wr2, Wl3, bl3, Wr3, Wl4, bl4, Wr4, fW1, fb1, fW2, fb2, fW3, fb3):
    raise NotImplementedError("write your pallas kernel here")


# speedup vs baseline: 4.6709x; 4.6709x over previous
import functools

import jax
import jax.numpy as jnp
from jax import lax
from jax.experimental import pallas as pl
from jax.experimental.pallas import tpu as pltpu
from jax.experimental.pallas import tpu_sc as plsc

_N = 10000
_NP = 10240
_E = 320000
_NC = 2
_NS = 16
_NW = _NC * _NS
_EPW = _E // _NW
_CHUNK = 80
_NITER = _EPW // _CHUNK
_RPS = _NP // _NS


def _make_sc_aggregate(dpad):
  mesh = plsc.VectorSubcoreMesh(core_axis_name="c", subcore_axis_name="s")

  @functools.partial(
      pl.kernel,
      mesh=mesh,
      compiler_params=pltpu.CompilerParams(use_tc_tiling_on_sc=False),
      out_type=jax.ShapeDtypeStruct((_NC, _NP, dpad), jnp.float32),
      scratch_types=[
          pltpu.VMEM((1, _CHUNK), jnp.int32),
          pltpu.VMEM((1, _CHUNK), jnp.int32),
          pltpu.VMEM((_CHUNK, dpad), jnp.float32),
          pltpu.VMEM_SHARED((_NP, dpad), jnp.float32),
          pltpu.SemaphoreType.DMA,
      ],
  )
  def agg(table_hbm, src_hbm, dst_hbm, z_hbm, out_hbm,
          src_v, dst_v, rows_v, acc_sh, sem):
    c = lax.axis_index("c")
    s = lax.axis_index("s")
    wid = c * _NS + s

    pltpu.sync_copy(z_hbm, acc_sh.at[pl.ds(s * _RPS, _RPS)])

    plsc.subcore_barrier()

    base = wid * _EPW

    @pl.loop(0, _NITER)
    def _(i):
      off = base + i * _CHUNK
      pltpu.sync_copy(src_hbm.at[pl.ds(off, _CHUNK)], src_v.at[0])
      pltpu.sync_copy(dst_hbm.at[pl.ds(off, _CHUNK)], dst_v.at[0])
      pltpu.async_copy(table_hbm.at[src_v.at[0]], rows_v, sem).wait()
      pltpu.sync_copy(rows_v, acc_sh.at[dst_v.at[0]], add=True)

    plsc.subcore_barrier()

    pltpu.sync_copy(acc_sh.at[pl.ds(s * _RPS, _RPS)],
                    out_hbm.at[c].at[pl.ds(s * _RPS, _RPS)])

  return agg


def _lrelu(x):
  return jnp.where(x >= 0, x, 0.01 * x)


def _bn(x):
  m = jnp.mean(x, axis=0, keepdims=True)
  v = jnp.mean((x - m) * (x - m), axis=0, keepdims=True)
  return (x - m) * lax.rsqrt(v + 1e-5)


def _dot(a, b):
  return jnp.dot(a, b, preferred_element_type=jnp.float32)


def _tc_layer1a(aggp, x, Wl1, bl1, Wr1):
  def body(aggp_ref, x_ref, wl_ref, bl_ref, wr_ref, z_ref, inv_ref):
    agg = aggp_ref[0][:_N] + aggp_ref[1][:_N]
    inv = 1.0 / jnp.maximum(agg[:, 128:129], 1.0)
    mean = agg[:, :128] * inv
    z = _dot(mean, wl_ref[...]) + bl_ref[...][None, :] + _dot(x_ref[...], wr_ref[...])
    z_ref[...] = _lrelu(z)
    inv_ref[...] = inv

  return pl.pallas_call(
      body,
      out_shape=[
          jax.ShapeDtypeStruct((_N, 320), jnp.float32),
          jax.ShapeDtypeStruct((_N, 1), jnp.float32),
      ],
  )(aggp, x, Wl1, bl1, Wr1)


def _tc_layer1b(z, Wl2, Wr2):
  def body(z_ref, wl_ref, wr_ref, ha_ref, hb_ref, xw_ref):
    y = _bn(z_ref[...])
    h = _dot(y, wl_ref[...])
    ha_ref[...] = h[:, :128]
    hb_ref[...] = jnp.pad(h[:, 128:], ((0, 0), (0, 12)))
    xw_ref[...] = _dot(y, wr_ref[...])

  return pl.pallas_call(
      body,
      out_shape=[
          jax.ShapeDtypeStruct((_N, 128), jnp.float32),
          jax.ShapeDtypeStruct((_N, 64), jnp.float32),
          jax.ShapeDtypeStruct((_N, 180), jnp.float32),
      ],
  )(z, Wl2, Wr2)


def _tc_layer2(aggpa, aggpb, xw, inv, bl, Wl_next, Wr_next):
  def body(aggpa_ref, aggpb_ref, xw_ref, inv_ref, bl_ref, wln_ref, wrn_ref,
           hp_ref, xwn_ref):
    agg = jnp.concatenate(
        [aggpa_ref[0][:_N] + aggpa_ref[1][:_N],
         aggpb_ref[0][:_N, :52] + aggpb_ref[1][:_N, :52]], axis=1)
    y = agg * inv_ref[...] + bl_ref[...][None, :] + xw_ref[...]
    y = _bn(_lrelu(y))
    h = _dot(y, wln_ref[...])
    hp_ref[...] = jnp.pad(h, ((0, 0), (0, 6)))
    xwn_ref[...] = _dot(y, wrn_ref[...])

  return pl.pallas_call(
      body,
      out_shape=[
          jax.ShapeDtypeStruct((_N, 96), jnp.float32),
          jax.ShapeDtypeStruct((_N, 90), jnp.float32),
      ],
  )(aggpa, aggpb, xw, inv, bl, Wl_next, Wr_next)


def _tc_layer3(aggp, xw, inv, bl, Wl_next, Wr_next):
  def body(aggp_ref, xw_ref, inv_ref, bl_ref, wln_ref, wrn_ref,
           hp_ref, xwn_ref):
    agg = aggp_ref[0][:_N, :90] + aggp_ref[1][:_N, :90]
    y = agg * inv_ref[...] + bl_ref[...][None, :] + xw_ref[...]
    y = _bn(_lrelu(y))
    h = _dot(y, wln_ref[...])
    hp_ref[...] = jnp.pad(h, ((0, 0), (0, 14)))
    xwn_ref[...] = _dot(y, wrn_ref[...])

  return pl.pallas_call(
      body,
      out_shape=[
          jax.ShapeDtypeStruct((_N, 64), jnp.float32),
          jax.ShapeDtypeStruct((_N, 50), jnp.float32),
      ],
  )(aggp, xw, inv, bl, Wl_next, Wr_next)


def _tc_layer4(aggp, xw, inv, bl4, fW1, fb1, fW2, fb2, fW3, fb3):
  blen = _N // 16

  def body(aggp_ref, xw_ref, inv_ref, bl_ref,
           fw1_ref, fb1_ref, fw2_ref, fb2_ref, fw3_ref, fb3_ref, out_ref):
    agg = aggp_ref[0][:_N, :50] + aggp_ref[1][:_N, :50]
    y = agg * inv_ref[...] + bl_ref[...][None, :] + xw_ref[...]
    y = _bn(_lrelu(y))
    col = lax.broadcasted_iota(jnp.int32, (16, _N), 1) // blen
    row = lax.broadcasted_iota(jnp.int32, (16, _N), 0)
    sel = (col == row).astype(jnp.float32)
    p = _dot(sel, y)
    p = _dot(p, fw1_ref[...]) + fb1_ref[...][None, :]
    p = _dot(p, fw2_ref[...]) + fb2_ref[...][None, :]
    p = _dot(p, fw3_ref[...]) + fb3_ref[...][None, :]
    out_ref[...] = p

  return pl.pallas_call(
      body,
      out_shape=jax.ShapeDtypeStruct((16, 1), jnp.float32),
  )(aggp, xw, inv, bl4, fW1, fb1, fW2, fb2, fW3, fb3)


_agg144 = _make_sc_aggregate(144)
_agg128 = _make_sc_aggregate(128)
_agg96 = _make_sc_aggregate(96)
_agg64 = _make_sc_aggregate(64)


def kernel(x_in, edge_index, Wl1, bl1, Wr1, Wl2, bl2, Wr2, Wl3, bl3, Wr3,
           Wl4, bl4, Wr4, fW1, fb1, fW2, fb2, fW3, fb3):
  src = edge_index[0]
  dst = edge_index[1]

  xp = jnp.concatenate(
      [x_in, jnp.ones((_N, 1), jnp.float32), jnp.zeros((_N, 15), jnp.float32)],
      axis=1)
  a1 = _agg144(xp, src, dst, jnp.zeros((_RPS, 144), jnp.float32))
  z1, inv = _tc_layer1a(a1, x_in, Wl1, bl1, Wr1)
  h2a, h2b, xw2 = _tc_layer1b(z1, Wl2, Wr2)

  a2a = _agg128(h2a, src, dst, jnp.zeros((_RPS, 128), jnp.float32))
  a2b = _agg64(h2b, src, dst, jnp.zeros((_RPS, 64), jnp.float32))
  h3p, xw3 = _tc_layer2(a2a, a2b, xw2, inv, bl2, Wl3, Wr3)

  a3 = _agg96(h3p, src, dst, jnp.zeros((_RPS, 96), jnp.float32))
  h4p, xw4 = _tc_layer3(a3, xw3, inv, bl3, Wl4, Wr4)

  a4 = _agg64(h4p, src, dst, jnp.zeros((_RPS, 64), jnp.float32))
  return _tc_layer4(a4, xw4, inv, bl4, fW1, fb1, fW2, fb2, fW3, fb3)
